# baseline (speedup 1.0000x reference)
import jax
import jax.numpy as jnp
import numpy as np
from jax import lax
from jax.experimental import pallas as pl
from jax.experimental.pallas import tpu as pltpu
from jax.experimental.pallas import tpu_sc as plsc

N_NODES = 100000
D = 128
B = 10000
K = 10
F = K + 1

NC, NS, L = 2, 16, 16
NW = NC * NS
B_PAD = 10240
BPW = B_PAD // NW
C = 8
STEPS = BPW // C
RPS = C * F
NVREG = D // L
NIDX = BPW * F
NBUF = 5
IDX_CHUNK = 80

_INV_DENOM = 1.0 / 11.0

_POS_NODE = np.arange(NIDX, dtype=np.int32) // F
_POS_SLOT = np.arange(NIDX, dtype=np.int32) % F


def _sc_body(nodes_hbm, neigh_hbm, x_hbm, pos_node_hbm, pos_slot_hbm, agg_hbm,
             nodes_v, neigh_v, pos_node_v, pos_slot_v, all_idx_v,
             rows0, rows1, rows2, rows3, rows4,
             sem0, sem1, sem2, sem3, sem4, out_v, sem_w, sem_a):
    bufs = (rows0, rows1, rows2, rows3, rows4)
    sems = (sem0, sem1, sem2, sem3, sem4)
    wid = lax.axis_index("s") * NC + lax.axis_index("c")
    base = wid * BPW

    pltpu.sync_copy(nodes_hbm.at[pl.ds(base, BPW)], nodes_v)
    pltpu.sync_copy(pos_node_hbm, pos_node_v)
    pltpu.sync_copy(pos_slot_hbm, pos_slot_v)

    pltpu.sync_copy(neigh_hbm.at[pl.ds(base * K, BPW * K)], neigh_v)

    def flatten(t, carry):
        prow = pos_node_v[pl.ds(t * L, L)]
        pslot = pos_slot_v[pl.ds(t * L, L)]
        self_ids = plsc.load_gather(nodes_v, [prow])
        npos = jnp.maximum(prow * K + pslot - 1, 0)
        nei_ids = plsc.load_gather(neigh_v, [npos])
        ids = jnp.where(pslot == 0, self_ids, nei_ids)
        all_idx_v[pl.ds(t * L, L)] = jnp.clip(ids, 0, N_NODES - 1)
        return carry

    lax.fori_loop(0, NIDX // L, flatten, 0)

    def issue(s, b):
        pltpu.async_copy(
            x_hbm.at[all_idx_v.at[pl.ds(s * RPS, RPS)]], bufs[b], sems[b])

    def drain(b):
        pltpu.make_async_copy(x_hbm.at[pl.ds(0, RPS)], bufs[b], sems[b]).wait()

    def compute(s, b):
        buf = bufs[b]

        def node(i, carry):
            for v in range(NVREG):
                acc = buf[i * F, pl.ds(v * L, L)]
                for j in range(1, F):
                    acc = acc + buf[i * F + j, pl.ds(v * L, L)]
                out_v[s * C + i, pl.ds(v * L, L)] = acc * _INV_DENOM
            return carry

        lax.fori_loop(0, C, node, 0)
        pltpu.async_copy(
            out_v.at[pl.ds(s * C, C)],
            agg_hbm.at[pl.ds(base + s * C, C)], sem_w)

    for b in range(NBUF - 1):
        issue(b, b)

    def body(t, carry):
        s0 = NBUF * t
        issue(s0 + NBUF - 1, NBUF - 1)
        for b in range(NBUF):
            drain(b)
            compute(s0 + b, b)
            if b < NBUF - 1:
                @pl.when(s0 + NBUF + b < STEPS)
                def _():
                    issue(s0 + NBUF + b, b)
        return carry

    lax.fori_loop(0, STEPS // NBUF, body, 0)

    pltpu.make_async_copy(
        out_v, agg_hbm.at[pl.ds(base, BPW)], sem_w).wait()


@jax.jit
def _sc_aggregate(nodes_pad, neigh_flat, x, pos_node, pos_slot):
    mesh = plsc.VectorSubcoreMesh(core_axis_name="c", subcore_axis_name="s")
    return pl.kernel(
        _sc_body,
        out_type=jax.ShapeDtypeStruct((B_PAD, D), jnp.float32),
        mesh=mesh,
        compiler_params=pltpu.CompilerParams(needs_layout_passes=False),
        scratch_types=[
            pltpu.VMEM((BPW,), jnp.int32),
            pltpu.VMEM((BPW * K,), jnp.int32),
            pltpu.VMEM((NIDX,), jnp.int32),
            pltpu.VMEM((NIDX,), jnp.int32),
            pltpu.VMEM((NIDX,), jnp.int32),
            pltpu.VMEM((RPS, D), jnp.float32),
            pltpu.VMEM((RPS, D), jnp.float32),
            pltpu.VMEM((RPS, D), jnp.float32),
            pltpu.VMEM((RPS, D), jnp.float32),
            pltpu.VMEM((RPS, D), jnp.float32),
            pltpu.SemaphoreType.DMA,
            pltpu.SemaphoreType.DMA,
            pltpu.SemaphoreType.DMA,
            pltpu.SemaphoreType.DMA,
            pltpu.SemaphoreType.DMA,
            pltpu.VMEM((BPW, D), jnp.float32),
            pltpu.SemaphoreType.DMA,
            pltpu.SemaphoreType.DMA,
        ],
    )(nodes_pad, neigh_flat, x, pos_node, pos_slot)


def _mm_body(a_ref, wt_ref, o_ref):
    o_ref[...] = jnp.maximum(
        jnp.dot(a_ref[...], wt_ref[...], preferred_element_type=jnp.float32),
        0.0)


MM_BLOCK = 400


@jax.jit
def _tc_matmul_relu(agg_pad, Wt):
    return pl.pallas_call(
        _mm_body,
        grid=(B // MM_BLOCK,),
        in_specs=[
            pl.BlockSpec((MM_BLOCK, D), lambda i: (i, 0)),
            pl.BlockSpec((D, D), lambda i: (0, 0)),
        ],
        out_specs=pl.BlockSpec((MM_BLOCK, D), lambda i: (i, 0)),
        out_shape=jax.ShapeDtypeStruct((B, D), jnp.float32),
    )(agg_pad, Wt)


def kernel(nodes, adj, x, W):
    nodes_pad = jnp.pad(nodes, (0, B_PAD - B))
    neigh_flat = jnp.take(adj, nodes_pad, axis=0).reshape(-1)
    agg_pad = _sc_aggregate(nodes_pad, neigh_flat, x, _POS_NODE, _POS_SLOT)
    return _tc_matmul_relu(agg_pad, W.T)
